# speedup vs baseline: 1.1696x; 1.1696x over previous
import functools

import jax
import jax.numpy as jnp
from jax import lax
from jax.experimental import pallas as pl
from jax.experimental.pallas import tpu as pltpu

D_MODEL = 4096
N_EXP = 64
K = 8
BLK_T = 512


def _router_body(x_ref, w_ref, b_ref, gates_ref, idx_ref):
    x_blk = x_ref[...]
    w = w_ref[...]
    logits = lax.dot_general(
        x_blk, w, (((1,), (1,)), ((), ())),
        preferred_element_type=jnp.float32,
    )
    logits = logits + b_ref[...]

    eiota = lax.broadcasted_iota(jnp.int32, (BLK_T, N_EXP), 1)
    neg_inf = jnp.float32(float("-inf"))

    vals = []
    idxs = []
    l = logits
    for _ in range(K):
        m = jnp.max(l, axis=1, keepdims=True)
        cand = jnp.where(l == m, eiota, N_EXP)
        a = jnp.min(cand, axis=1, keepdims=True)
        vals.append(m)
        idxs.append(a)
        l = jnp.where(eiota == a, neg_inf, l)

    v = jnp.concatenate(vals, axis=1)
    e = jnp.exp(v - vals[0])
    g = e / jnp.sum(e, axis=1, keepdims=True)
    gates_ref[...] = g
    idx_ref[...] = jnp.concatenate(idxs, axis=1).astype(jnp.int32)


@jax.jit
def kernel(x, W, b):
    B, S, D = x.shape
    T = B * S
    xf = x.reshape(T, D)
    b2 = b.reshape(1, N_EXP)
    grid = (T // BLK_T,)
    gates, idx = pl.pallas_call(
        _router_body,
        grid=grid,
        in_specs=[
            pl.BlockSpec((BLK_T, D), lambda i: (i, 0)),
            pl.BlockSpec((N_EXP, D), lambda i: (0, 0)),
            pl.BlockSpec((1, N_EXP), lambda i: (0, 0)),
        ],
        out_specs=[
            pl.BlockSpec((BLK_T, K), lambda i: (i, 0)),
            pl.BlockSpec((BLK_T, K), lambda i: (i, 0)),
        ],
        out_shape=[
            jax.ShapeDtypeStruct((T, K), jnp.float32),
            jax.ShapeDtypeStruct((T, K), jnp.int32),
        ],
        compiler_params=pltpu.CompilerParams(
            dimension_semantics=("arbitrary",),
        ),
    )(xf, W, b2)
    return gates.reshape(B, S, K), idx.reshape(B, S, K)


# speedup vs baseline: 1.3036x; 1.1146x over previous
import functools

import jax
import jax.numpy as jnp
from jax import lax
from jax.experimental import pallas as pl
from jax.experimental.pallas import tpu as pltpu

D_MODEL = 4096
N_EXP = 64
K = 8
BLK_T = 512


def _router_body(x_ref, w_ref, b_ref, gates_ref, idx_ref):
    x_blk = x_ref[...]
    w = w_ref[...]
    logits = lax.dot_general(
        x_blk, w, (((1,), (1,)), ((), ())),
        preferred_element_type=jnp.float32,
    )
    logits = logits + b_ref[...]

    fiota = lax.broadcasted_iota(jnp.int32, (BLK_T, N_EXP), 1).astype(jnp.float32)
    neg_inf = jnp.float32(float("-inf"))
    big = jnp.float32(N_EXP)

    vals = []
    idxs = []
    l = logits
    for _ in range(K):
        m = jnp.max(l, axis=1, keepdims=True)
        cand = jnp.where(l == m, fiota, big)
        a = jnp.min(cand, axis=1, keepdims=True)
        vals.append(m)
        idxs.append(a)
        l = jnp.where(fiota == a, neg_inf, l)

    v = jnp.concatenate(vals, axis=1)
    e = jnp.exp(v - vals[0])
    g = e / jnp.sum(e, axis=1, keepdims=True)
    gates_ref[...] = g
    idx_ref[...] = jnp.concatenate(idxs, axis=1).astype(jnp.int32)


@jax.jit
def kernel(x, W, b):
    B, S, D = x.shape
    T = B * S
    xf = x.reshape(T, D)
    b2 = b.reshape(1, N_EXP)
    grid = (T // BLK_T,)
    gates, idx = pl.pallas_call(
        _router_body,
        grid=grid,
        in_specs=[
            pl.BlockSpec((BLK_T, D), lambda i: (i, 0)),
            pl.BlockSpec((N_EXP, D), lambda i: (0, 0)),
            pl.BlockSpec((1, N_EXP), lambda i: (0, 0)),
        ],
        out_specs=[
            pl.BlockSpec((BLK_T, K), lambda i: (i, 0)),
            pl.BlockSpec((BLK_T, K), lambda i: (i, 0)),
        ],
        out_shape=[
            jax.ShapeDtypeStruct((T, K), jnp.float32),
            jax.ShapeDtypeStruct((T, K), jnp.int32),
        ],
        compiler_params=pltpu.CompilerParams(
            dimension_semantics=("arbitrary",),
        ),
    )(xf, W, b2)
    return gates.reshape(B, S, K), idx.reshape(B, S, K)


# speedup vs baseline: 1.4491x; 1.1116x over previous
import functools

import jax
import jax.numpy as jnp
from jax import lax
from jax.experimental import pallas as pl
from jax.experimental.pallas import tpu as pltpu

D_MODEL = 4096
N_EXP = 64
K = 8
BLK_T = 1024


def _router_body(x_ref, w_ref, b_ref, gates_ref, idx_ref):
    x_blk = x_ref[...]
    w = w_ref[...]
    logits = lax.dot_general(
        x_blk, w, (((1,), (1,)), ((), ())),
        preferred_element_type=jnp.float32,
    )
    logits = logits + b_ref[...]

    fiota = lax.broadcasted_iota(jnp.int32, (BLK_T, N_EXP), 1).astype(jnp.float32)
    neg_inf = jnp.float32(float("-inf"))
    big = jnp.float32(N_EXP)

    vals = []
    idxs = []
    l = logits
    for _ in range(K):
        m = jnp.max(l, axis=1, keepdims=True)
        cand = jnp.where(l == m, fiota, big)
        a = jnp.min(cand, axis=1, keepdims=True)
        vals.append(m)
        idxs.append(a)
        l = jnp.where(fiota == a, neg_inf, l)

    v = jnp.concatenate(vals, axis=1)
    e = jnp.exp(v - vals[0])
    g = e / jnp.sum(e, axis=1, keepdims=True)
    gates_ref[...] = g
    idx_ref[...] = jnp.concatenate(idxs, axis=1).astype(jnp.int32)


@jax.jit
def kernel(x, W, b):
    B, S, D = x.shape
    T = B * S
    xf = x.reshape(T, D)
    b2 = b.reshape(1, N_EXP)
    grid = (T // BLK_T,)
    gates, idx = pl.pallas_call(
        _router_body,
        grid=grid,
        in_specs=[
            pl.BlockSpec((BLK_T, D), lambda i: (i, 0)),
            pl.BlockSpec((N_EXP, D), lambda i: (0, 0)),
            pl.BlockSpec((1, N_EXP), lambda i: (0, 0)),
        ],
        out_specs=[
            pl.BlockSpec((BLK_T, K), lambda i: (i, 0)),
            pl.BlockSpec((BLK_T, K), lambda i: (i, 0)),
        ],
        out_shape=[
            jax.ShapeDtypeStruct((T, K), jnp.float32),
            jax.ShapeDtypeStruct((T, K), jnp.int32),
        ],
        compiler_params=pltpu.CompilerParams(
            dimension_semantics=("arbitrary",),
        ),
    )(xf, W, b2)
    return gates.reshape(B, S, K), idx.reshape(B, S, K)
